# Initial kernel scaffold; baseline (speedup 1.0000x reference)
import jax
import jax.numpy as jnp
from jax.experimental import pallas as pl


def kernel(xyz, points, W0, b0, W1, b1, W2, b2):
    raise NotImplementedError("write your pallas kernel here")


# speedup vs baseline: 14.1359x; 14.1359x over previous
import functools

import jax
import jax.numpy as jnp
from jax import lax
from jax.experimental import pallas as pl
from jax.experimental.pallas import tpu as pltpu
from jax.experimental.pallas import tpu_sc as plsc

_B, _A, _C = 8, 4096, 64
_N, _S = 512, 32
_D = 128
_R2 = 0.2 * 0.2
_NB = 256
_NSPLIT = _N // _NB


def _fps_body(xt_ref, new_xyz_ref):
    X = xt_ref[0]
    Y = xt_ref[1]
    Z = xt_ref[2]
    col = lax.broadcasted_iota(jnp.int32, (_B, _A), 1)

    def body(i, carry):
        dists, far = carry
        onehot = col == far
        cx = jnp.sum(jnp.where(onehot, X, 0.0), axis=1, keepdims=True)
        cy = jnp.sum(jnp.where(onehot, Y, 0.0), axis=1, keepdims=True)
        cz = jnp.sum(jnp.where(onehot, Z, 0.0), axis=1, keepdims=True)
        new_xyz_ref[:, pl.ds(i, 1), :] = jnp.concatenate(
            [cx, cy, cz], axis=1).reshape(_B, 1, 3)
        d = (X - cx) ** 2 + (Y - cy) ** 2 + (Z - cz) ** 2
        dists = jnp.minimum(dists, d)
        m = jnp.max(dists, axis=1, keepdims=True)
        far = jnp.min(jnp.where(dists == m, col, _A), axis=1, keepdims=True)
        return dists, far

    dists0 = jnp.full((_B, _A), 1e10, dtype=jnp.float32)
    far0 = jnp.zeros((_B, 1), dtype=jnp.int32)
    lax.fori_loop(0, _N, body, (dists0, far0))


def _fps(xt):
    return pl.pallas_call(
        _fps_body,
        out_shape=jax.ShapeDtypeStruct((_B, _N, 3), jnp.float32),
    )(xt)


def _ballq_body(xtb_ref, nxyz_ref, idx_ref):
    b = pl.program_id(0)
    xtb = xtb_ref[0]
    X = xtb[0:1]
    Y = xtb[1:2]
    Z = xtb[2:3]
    na = nxyz_ref[0]
    nx = na[:, 0:1]
    ny = na[:, 1:2]
    nz = na[:, 2:3]
    d2 = (nx - X) ** 2 + (ny - Y) ** 2 + (nz - Z) ** 2
    mask = d2 <= jnp.float32(_R2)
    col = lax.broadcasted_iota(jnp.int32, (_NB, _A), 1)
    cols = []
    first = None
    for s in range(_S):
        cand = jnp.where(mask, col, _A)
        f = jnp.min(cand, axis=1, keepdims=True)
        if s == 0:
            first = f
            sel = f
        else:
            sel = jnp.where(f == _A, first, f)
        cols.append(sel)
        mask = jnp.logical_and(mask, col != f)
    idx = jnp.concatenate(cols, axis=1) + b * _A
    idx_ref[0, 0] = idx.reshape(1, 1, _NB, _S)[0, 0]


def _ballq(xtb, new_xyz):
    return pl.pallas_call(
        _ballq_body,
        grid=(_B, _NSPLIT),
        in_specs=[
            pl.BlockSpec((1, 3, _A), lambda b, h: (b, 0, 0)),
            pl.BlockSpec((1, _NB, 3), lambda b, h: (b, h, 0)),
        ],
        out_specs=pl.BlockSpec((1, 1, _NB, _S), lambda b, h: (b, h, 0, 0)),
        out_shape=jax.ShapeDtypeStruct((_B, _NSPLIT, _NB, _S), jnp.int32),
    )(xtb, new_xyz)


def _feat_body(xyz_ref, pts_ref, nxyz_ref, w0x_ref, w0p_ref, f_ref, c0_ref):
    w0x = w0x_ref[...]
    f = jnp.dot(xyz_ref[0], w0x, preferred_element_type=jnp.float32)
    f = f + jnp.dot(pts_ref[0], w0p_ref[...],
                    preferred_element_type=jnp.float32)
    f_ref[0] = f
    c0_ref[0] = jnp.dot(nxyz_ref[0], w0x, preferred_element_type=jnp.float32)


def _feat(xyz, points, new_xyz, w0x, w0p):
    return pl.pallas_call(
        _feat_body,
        grid=(_B,),
        in_specs=[
            pl.BlockSpec((1, _A, 3), lambda b: (b, 0, 0)),
            pl.BlockSpec((1, _A, _C), lambda b: (b, 0, 0)),
            pl.BlockSpec((1, _N, 3), lambda b: (b, 0, 0)),
            pl.BlockSpec((3, _D), lambda b: (0, 0)),
            pl.BlockSpec((_C, _D), lambda b: (0, 0)),
        ],
        out_specs=[
            pl.BlockSpec((1, _A, _D), lambda b: (b, 0, 0)),
            pl.BlockSpec((1, _N, _D), lambda b: (b, 0, 0)),
        ],
        out_shape=[
            jax.ShapeDtypeStruct((_B, _A, _D), jnp.float32),
            jax.ShapeDtypeStruct((_B, _N, _D), jnp.float32),
        ],
    )(xyz, points, new_xyz, w0x, w0p)


_NROWS = _B * _N * _S
_NW = 32
_CH = 128
_PER_W = _NROWS // _NW
_NCH = _PER_W // _CH


def _sc_gather(f2, idxg):
    mesh = plsc.VectorSubcoreMesh(core_axis_name="c", subcore_axis_name="s")

    @functools.partial(
        pl.kernel,
        out_type=jax.ShapeDtypeStruct((_NROWS, _D), jnp.float32),
        mesh=mesh,
        scratch_types=[
            pltpu.VMEM((_CH,), jnp.int32),
            pltpu.VMEM((_CH, _D), jnp.float32),
            pltpu.SemaphoreType.DMA,
        ],
    )
    def k(f_hbm, idx_hbm, out_hbm, idx_v, rows_v, sem):
        wid = lax.axis_index("s") * 2 + lax.axis_index("c")

        def body(j, _):
            base = wid * _PER_W + j * _CH
            pltpu.sync_copy(idx_hbm.at[pl.ds(base, _CH)], idx_v)
            pltpu.async_copy(f_hbm.at[idx_v], rows_v, sem).wait()
            pltpu.sync_copy(rows_v, out_hbm.at[pl.ds(base, _CH)])
            return 0

        lax.fori_loop(0, _NCH, body, 0)

    return k(f2, idxg)


def _mlp_body(g_ref, c0_ref, b0_ref, w1_ref, b1_ref, w2_ref, b2_ref, out_ref):
    adj = b0_ref[...] - c0_ref[0]
    g = g_ref[...].reshape(_NB, _S, _D)
    h = jnp.maximum(g + adj[:, None, :], 0.0).reshape(_NB * _S, _D)
    h = jnp.dot(h, w1_ref[...], preferred_element_type=jnp.float32)
    h = jnp.maximum(h + b1_ref[...], 0.0)
    h = jnp.dot(h, w2_ref[...], preferred_element_type=jnp.float32)
    h = jnp.maximum(h + b2_ref[...], 0.0)
    out_ref[0] = jnp.max(h.reshape(_NB, _S, 128), axis=1)


def _mlp(g, c0, b0, w1, b1, w2, b2):
    return pl.pallas_call(
        _mlp_body,
        grid=(_B, _NSPLIT),
        in_specs=[
            pl.BlockSpec((_NB * _S, _D), lambda b, h: (b * _NSPLIT + h, 0)),
            pl.BlockSpec((1, _NB, _D), lambda b, h: (b, h, 0)),
            pl.BlockSpec((1, _D), lambda b, h: (0, 0)),
            pl.BlockSpec((_D, _C), lambda b, h: (0, 0)),
            pl.BlockSpec((1, _C), lambda b, h: (0, 0)),
            pl.BlockSpec((_C, 128), lambda b, h: (0, 0)),
            pl.BlockSpec((1, 128), lambda b, h: (0, 0)),
        ],
        out_specs=pl.BlockSpec((1, _NB, 128), lambda b, h: (b, h, 0)),
        out_shape=jax.ShapeDtypeStruct((_B, _N, 128), jnp.float32),
    )(g, c0, b0, w1, b1, w2, b2)


def kernel(xyz, points, W0, b0, W1, b1, W2, b2):
    xt = jnp.transpose(xyz, (2, 0, 1))
    new_xyz = _fps(xt)
    xtb = jnp.transpose(xyz, (0, 2, 1))
    idx = _ballq(xtb, new_xyz)
    w0pad = jnp.pad(W0, ((0, 0), (0, _D - _C)))
    f, c0 = _feat(xyz, points, new_xyz, w0pad[:3], w0pad[3:])
    g = _sc_gather(f.reshape(_B * _A, _D), idx.reshape(_NROWS))
    w1pad = jnp.pad(W1, ((0, _D - _C), (0, 0)))
    new_points = _mlp(g, c0, jnp.pad(b0, (0, _D - _C)).reshape(1, _D),
                      w1pad, b1.reshape(1, _C), W2, b2.reshape(1, 128))
    return new_xyz, new_points
